# baseline (speedup 1.0000x reference)
import functools

import jax
import jax.numpy as jnp
from jax import lax
from jax.experimental import pallas as pl
from jax.experimental.pallas import tpu as pltpu
from jax.experimental.pallas import tpu_sc as plsc

_S = 8
_D = 1024
_H1 = 512
_H2 = 256
_N = 8192

_TILE = 256
_NT = _N // _TILE + _S
_NPAD = _NT * _TILE

_NC = 2
_NS = 16
_NW = _NC * _NS
_BPW = _NPAD // _NW
_CHUNK = 32
_NCH = _BPW // _CHUNK


def _routing(species):
    species = species.astype(jnp.int32)
    oh = (species[:, None] == jnp.arange(_S, dtype=jnp.int32)[None, :])
    ranks = jnp.cumsum(oh.astype(jnp.int32), axis=0)
    counts = ranks[-1].astype(jnp.int32)
    myrank = jnp.take_along_axis(ranks, species[:, None], axis=1)[:, 0] - 1
    ptiles = (counts + _TILE - 1) // _TILE
    pstart = (jnp.cumsum(ptiles) - ptiles).astype(jnp.int32)
    t = jnp.arange(_NT, dtype=jnp.int32)
    ts = jnp.sum((t[:, None] >= pstart[None, :]).astype(jnp.int32), axis=1) - 1
    ts = jnp.clip(ts, 0, _S - 1).astype(jnp.int32)
    dest = pstart[species] * _TILE + myrank
    g = jnp.zeros((_NPAD,), jnp.int32).at[dest].set(
        jnp.arange(_N, dtype=jnp.int32))
    return ts, pstart, counts, g


def _sc_gather(aev, idx):
    mesh = plsc.VectorSubcoreMesh(core_axis_name="c", subcore_axis_name="s")

    @functools.partial(
        pl.kernel,
        out_type=jax.ShapeDtypeStruct((_NPAD, _D), jnp.float32),
        mesh=mesh,
        scratch_types=[
            pltpu.VMEM((_NCH, _CHUNK), jnp.int32),
            pltpu.VMEM((3, _CHUNK, _D), jnp.float32),
            pltpu.SemaphoreType.DMA,
            pltpu.SemaphoreType.DMA,
            pltpu.SemaphoreType.DMA,
            pltpu.SemaphoreType.DMA,
            pltpu.SemaphoreType.DMA,
            pltpu.SemaphoreType.DMA,
        ],
    )
    def gather_kernel(aev_hbm, idx_hbm, out_hbm, idx_v, rows_v,
                      g0, g1, g2, s0, s1, s2):
        gsems = (g0, g1, g2)
        ssems = (s0, s1, s2)
        wid = lax.axis_index("s") * _NC + lax.axis_index("c")
        pltpu.sync_copy(idx_hbm.at[wid], idx_v)
        base = wid * _BPW
        gat = [None] * _NCH
        sca = [None] * _NCH
        for c in range(min(2, _NCH)):
            gat[c] = pltpu.async_copy(
                aev_hbm.at[idx_v.at[c]], rows_v.at[c % 3], gsems[c % 3])
        for c in range(_NCH):
            gat[c].wait()
            sca[c] = pltpu.async_copy(
                rows_v.at[c % 3],
                out_hbm.at[pl.ds(base + c * _CHUNK, _CHUNK)], ssems[c % 3])
            if c + 2 < _NCH:
                if c >= 1:
                    sca[c - 1].wait()
                gat[c + 2] = pltpu.async_copy(
                    aev_hbm.at[idx_v.at[c + 2]], rows_v.at[(c + 2) % 3],
                    gsems[(c + 2) % 3])
        for c in range(max(_NCH - 3, 0), _NCH):
            sca[c].wait()

    return gather_kernel(aev, idx.reshape(_NW, _NCH, _CHUNK))


def _mlp_body(ts_ref, pstart_ref, counts_ref, b3_ref,
              x_ref, W1_ref, b1_ref, W2_ref, b2_ref, w3_ref, out_ref):
    i = pl.program_id(0)
    s = ts_ref[i]
    h = jnp.dot(x_ref[...], W1_ref[0], preferred_element_type=jnp.float32)
    h = h + b1_ref[0]
    h = jnp.where(h > 0.0, h, jnp.exp(h) - 1.0)
    h = jnp.dot(h, W2_ref[0], preferred_element_type=jnp.float32)
    h = h + b2_ref[0]
    h = jnp.where(h > 0.0, h, jnp.exp(h) - 1.0)
    row = lax.broadcasted_iota(jnp.int32, (_TILE, 1), 0)
    off0 = (i - pstart_ref[s]) * _TILE
    valid = ((row + off0) < counts_ref[s]).astype(jnp.float32)
    hsum = jnp.sum(h * valid, axis=0, keepdims=True)
    tile_e = jnp.sum(hsum * w3_ref[0]) + b3_ref[s] * jnp.sum(valid)

    @pl.when(i == 0)
    def _init():
        out_ref[0, 0] = 0.0

    out_ref[0, 0] += tile_e


def _mlp_call(xg, ts, pstart, counts, W1, b1, W2, b2, W3, b3):
    grid_spec = pltpu.PrefetchScalarGridSpec(
        num_scalar_prefetch=4,
        grid=(_NT,),
        in_specs=[
            pl.BlockSpec((_TILE, _D), lambda i, ts, ps, cn, b3: (i, 0)),
            pl.BlockSpec((1, _D, _H1), lambda i, ts, ps, cn, b3: (ts[i], 0, 0)),
            pl.BlockSpec((1, 1, _H1), lambda i, ts, ps, cn, b3: (ts[i], 0, 0)),
            pl.BlockSpec((1, _H1, _H2), lambda i, ts, ps, cn, b3: (ts[i], 0, 0)),
            pl.BlockSpec((1, 1, _H2), lambda i, ts, ps, cn, b3: (ts[i], 0, 0)),
            pl.BlockSpec((1, 1, _H2), lambda i, ts, ps, cn, b3: (ts[i], 0, 0)),
        ],
        out_specs=pl.BlockSpec(memory_space=pltpu.SMEM),
    )
    return pl.pallas_call(
        _mlp_body,
        grid_spec=grid_spec,
        out_shape=jax.ShapeDtypeStruct((1, 1), jnp.float32),
    )(ts, pstart, counts, b3.reshape(_S),
      xg, W1, b1.reshape(_S, 1, _H1), W2, b2.reshape(_S, 1, _H2),
      W3.reshape(_S, 1, _H2))


def kernel(aev, species, W1, b1, W2, b2, W3, b3):
    ts, pstart, counts, g = _routing(species)
    xg = _sc_gather(aev, g)
    total = _mlp_call(xg, ts, pstart, counts, W1, b1, W2, b2, W3, b3)
    return total.reshape(1)


# speedup vs baseline: 2.3992x; 2.3992x over previous
import functools

import jax
import jax.numpy as jnp
from jax import lax
from jax.experimental import pallas as pl
from jax.experimental.pallas import tpu as pltpu
from jax.experimental.pallas import tpu_sc as plsc

_S = 8
_D = 1024
_H1 = 512
_H2 = 256
_N = 8192

_TILE = 256
_NT = _N // _TILE + _S
_NPAD = _NT * _TILE

_NC = 2
_NS = 16
_NW = _NC * _NS
_BPW = _N // _NW
_CHUNK = 32
_NCH = _BPW // _CHUNK


def _routing(species):
    species = species.astype(jnp.int32)
    oh = (species[:, None] == jnp.arange(_S, dtype=jnp.int32)[None, :])
    ranks = jnp.cumsum(oh.astype(jnp.int32), axis=0)
    counts = ranks[-1].astype(jnp.int32)
    myrank = jnp.take_along_axis(ranks, species[:, None], axis=1)[:, 0] - 1
    ptiles = (counts + _TILE - 1) // _TILE
    pstart = (jnp.cumsum(ptiles) - ptiles).astype(jnp.int32)
    t = jnp.arange(_NT, dtype=jnp.int32)
    ts = jnp.sum((t[:, None] >= pstart[None, :]).astype(jnp.int32), axis=1) - 1
    ts = jnp.clip(ts, 0, _S - 1).astype(jnp.int32)
    dest = pstart[species] * _TILE + myrank
    return ts, pstart, counts, dest


def _sc_scatter(aev, dest):
    mesh = plsc.VectorSubcoreMesh(core_axis_name="c", subcore_axis_name="s")

    @functools.partial(
        pl.kernel,
        out_type=jax.ShapeDtypeStruct((_NPAD, _D), jnp.float32),
        mesh=mesh,
        scratch_types=[
            pltpu.VMEM((_NCH, _CHUNK), jnp.int32),
            pltpu.VMEM((3, _CHUNK, _D), jnp.float32),
            pltpu.SemaphoreType.DMA,
            pltpu.SemaphoreType.DMA,
            pltpu.SemaphoreType.DMA,
            pltpu.SemaphoreType.DMA,
            pltpu.SemaphoreType.DMA,
            pltpu.SemaphoreType.DMA,
        ],
    )
    def scatter_kernel(aev_hbm, dest_hbm, out_hbm, idx_v, rows_v,
                       g0, g1, g2, s0, s1, s2):
        gsems = (g0, g1, g2)
        ssems = (s0, s1, s2)
        wid = lax.axis_index("s") * _NC + lax.axis_index("c")
        pltpu.sync_copy(dest_hbm.at[wid], idx_v)
        base = wid * _BPW
        rd = [None] * _NCH
        wr = [None] * _NCH
        for c in range(min(2, _NCH)):
            rd[c] = pltpu.async_copy(
                aev_hbm.at[pl.ds(base + c * _CHUNK, _CHUNK)],
                rows_v.at[c % 3], gsems[c % 3])
        for c in range(_NCH):
            rd[c].wait()
            wr[c] = pltpu.async_copy(
                rows_v.at[c % 3], out_hbm.at[idx_v.at[c]], ssems[c % 3])
            if c + 2 < _NCH:
                if c >= 1:
                    wr[c - 1].wait()
                rd[c + 2] = pltpu.async_copy(
                    aev_hbm.at[pl.ds(base + (c + 2) * _CHUNK, _CHUNK)],
                    rows_v.at[(c + 2) % 3], gsems[(c + 2) % 3])
        for c in range(max(_NCH - 3, 0), _NCH):
            wr[c].wait()

    return scatter_kernel(aev, dest.reshape(_NW, _NCH, _CHUNK))


def _mlp_body(ts_ref, pstart_ref, counts_ref, b3_ref,
              x_ref, W1_ref, b1_ref, W2_ref, b2_ref, w3_ref, out_ref):
    i = pl.program_id(0)
    s = ts_ref[i]
    row = lax.broadcasted_iota(jnp.int32, (_TILE, 1), 0)
    off0 = (i - pstart_ref[s]) * _TILE
    valid_b = (row + off0) < counts_ref[s]
    x = jnp.where(valid_b, x_ref[...], 0.0)
    h = jnp.dot(x, W1_ref[0], preferred_element_type=jnp.float32)
    h = h + b1_ref[0]
    h = jnp.where(h > 0.0, h, jnp.exp(h) - 1.0)
    h = jnp.dot(h, W2_ref[0], preferred_element_type=jnp.float32)
    h = h + b2_ref[0]
    h = jnp.where(h > 0.0, h, jnp.exp(h) - 1.0)
    valid = valid_b.astype(jnp.float32)
    hsum = jnp.sum(h * valid, axis=0, keepdims=True)
    tile_e = jnp.sum(hsum * w3_ref[0]) + b3_ref[s] * jnp.sum(valid)

    @pl.when(i == 0)
    def _init():
        out_ref[0, 0] = 0.0

    out_ref[0, 0] += tile_e


def _mlp_call(xg, ts, pstart, counts, W1, b1, W2, b2, W3, b3):
    grid_spec = pltpu.PrefetchScalarGridSpec(
        num_scalar_prefetch=4,
        grid=(_NT,),
        in_specs=[
            pl.BlockSpec((_TILE, _D), lambda i, ts, ps, cn, b3: (i, 0)),
            pl.BlockSpec((1, _D, _H1), lambda i, ts, ps, cn, b3: (ts[i], 0, 0)),
            pl.BlockSpec((1, 1, _H1), lambda i, ts, ps, cn, b3: (ts[i], 0, 0)),
            pl.BlockSpec((1, _H1, _H2), lambda i, ts, ps, cn, b3: (ts[i], 0, 0)),
            pl.BlockSpec((1, 1, _H2), lambda i, ts, ps, cn, b3: (ts[i], 0, 0)),
            pl.BlockSpec((1, 1, _H2), lambda i, ts, ps, cn, b3: (ts[i], 0, 0)),
        ],
        out_specs=pl.BlockSpec(memory_space=pltpu.SMEM),
    )
    return pl.pallas_call(
        _mlp_body,
        grid_spec=grid_spec,
        out_shape=jax.ShapeDtypeStruct((1, 1), jnp.float32),
    )(ts, pstart, counts, b3.reshape(_S),
      xg, W1, b1.reshape(_S, 1, _H1), W2, b2.reshape(_S, 1, _H2),
      W3.reshape(_S, 1, _H2))


def kernel(aev, species, W1, b1, W2, b2, W3, b3):
    ts, pstart, counts, dest = _routing(species)
    xg = _sc_scatter(aev, dest)
    total = _mlp_call(xg, ts, pstart, counts, W1, b1, W2, b2, W3, b3)
    return total.reshape(1)


# speedup vs baseline: 2.7373x; 1.1409x over previous
import functools

import jax
import jax.numpy as jnp
from jax import lax
from jax.experimental import pallas as pl
from jax.experimental.pallas import tpu as pltpu
from jax.experimental.pallas import tpu_sc as plsc

_S = 8
_D = 1024
_H1 = 512
_H2 = 256
_N = 8192

_TILE = 256
_NT = _N // _TILE + _S
_NPAD = _NT * _TILE

_NC = 2
_NS = 16
_NW = _NC * _NS
_BPW = _N // _NW
_CHUNK = 32
_NCH = _BPW // _CHUNK

_BLK = 128
_NB = _N // _BLK


def _routing(species):
    species = species.astype(jnp.int32)
    oh = (species[:, None] == jnp.arange(_S, dtype=jnp.int32)[None, :])
    ohf = oh.astype(jnp.float32)
    ohb = ohf.reshape(_NB, _BLK, _S)
    ltri = jnp.tril(jnp.ones((_BLK, _BLK), jnp.float32))
    within = jnp.einsum('ij,bjs->bis', ltri, ohb,
                        preferred_element_type=jnp.float32)
    bsum = jnp.sum(ohb, axis=1)
    bpref = jnp.cumsum(bsum, axis=0) - bsum
    ranks = (within + bpref[:, None, :]).reshape(_N, _S)
    counts = (bpref[-1] + bsum[-1]).astype(jnp.int32)
    myrank = jnp.sum(ranks * ohf, axis=1) - 1.0
    ptiles = (counts + _TILE - 1) // _TILE
    pstart = (jnp.cumsum(ptiles) - ptiles).astype(jnp.int32)
    t = jnp.arange(_NT, dtype=jnp.int32)
    ts = jnp.sum((t[:, None] >= pstart[None, :]).astype(jnp.int32), axis=1) - 1
    ts = jnp.clip(ts, 0, _S - 1).astype(jnp.int32)
    seg0 = jnp.sum(ohf * pstart.astype(jnp.float32)[None, :], axis=1)
    dest = (seg0 * _TILE + myrank).astype(jnp.int32)
    return ts, pstart, counts, dest


def _sc_scatter(aev, dest):
    mesh = plsc.VectorSubcoreMesh(core_axis_name="c", subcore_axis_name="s")

    @functools.partial(
        pl.kernel,
        out_type=jax.ShapeDtypeStruct((_NPAD, _D), jnp.float32),
        mesh=mesh,
        scratch_types=[
            pltpu.VMEM((_NCH, _CHUNK), jnp.int32),
            pltpu.VMEM((3, _CHUNK, _D), jnp.float32),
            pltpu.SemaphoreType.DMA,
            pltpu.SemaphoreType.DMA,
            pltpu.SemaphoreType.DMA,
            pltpu.SemaphoreType.DMA,
            pltpu.SemaphoreType.DMA,
            pltpu.SemaphoreType.DMA,
        ],
    )
    def scatter_kernel(aev_hbm, dest_hbm, out_hbm, idx_v, rows_v,
                       g0, g1, g2, s0, s1, s2):
        gsems = (g0, g1, g2)
        ssems = (s0, s1, s2)
        wid = lax.axis_index("s") * _NC + lax.axis_index("c")
        pltpu.sync_copy(dest_hbm.at[wid], idx_v)
        base = wid * _BPW
        rd = [None] * _NCH
        wr = [None] * _NCH
        for c in range(min(2, _NCH)):
            rd[c] = pltpu.async_copy(
                aev_hbm.at[pl.ds(base + c * _CHUNK, _CHUNK)],
                rows_v.at[c % 3], gsems[c % 3])
        for c in range(_NCH):
            rd[c].wait()
            wr[c] = pltpu.async_copy(
                rows_v.at[c % 3], out_hbm.at[idx_v.at[c]], ssems[c % 3])
            if c + 2 < _NCH:
                if c >= 1:
                    wr[c - 1].wait()
                rd[c + 2] = pltpu.async_copy(
                    aev_hbm.at[pl.ds(base + (c + 2) * _CHUNK, _CHUNK)],
                    rows_v.at[(c + 2) % 3], gsems[(c + 2) % 3])
        for c in range(max(_NCH - 3, 0), _NCH):
            wr[c].wait()

    return scatter_kernel(aev, dest.reshape(_NW, _NCH, _CHUNK))


def _mlp_body(ts_ref, pstart_ref, counts_ref, b3_ref,
              x_ref, W1_ref, b1_ref, W2_ref, b2_ref, w3_ref, out_ref):
    i = pl.program_id(0)
    s = ts_ref[i]
    row = lax.broadcasted_iota(jnp.int32, (_TILE, 1), 0)
    off0 = (i - pstart_ref[s]) * _TILE
    valid_b = (row + off0) < counts_ref[s]
    x = jnp.where(valid_b, x_ref[...], 0.0)
    h = jnp.dot(x, W1_ref[0], preferred_element_type=jnp.float32)
    h = h + b1_ref[0]
    h = jnp.where(h > 0.0, h, jnp.exp(h) - 1.0)
    h = jnp.dot(h, W2_ref[0], preferred_element_type=jnp.float32)
    h = h + b2_ref[0]
    h = jnp.where(h > 0.0, h, jnp.exp(h) - 1.0)
    valid = valid_b.astype(jnp.float32)
    hsum = jnp.sum(h * valid, axis=0, keepdims=True)
    tile_e = jnp.sum(hsum * w3_ref[0]) + b3_ref[s] * jnp.sum(valid)

    @pl.when(i == 0)
    def _init():
        out_ref[0, 0] = 0.0

    out_ref[0, 0] += tile_e


def _mlp_call(xg, ts, pstart, counts, W1, b1, W2, b2, W3, b3):
    grid_spec = pltpu.PrefetchScalarGridSpec(
        num_scalar_prefetch=4,
        grid=(_NT,),
        in_specs=[
            pl.BlockSpec((_TILE, _D), lambda i, ts, ps, cn, b3: (i, 0)),
            pl.BlockSpec((1, _D, _H1), lambda i, ts, ps, cn, b3: (ts[i], 0, 0)),
            pl.BlockSpec((1, 1, _H1), lambda i, ts, ps, cn, b3: (ts[i], 0, 0)),
            pl.BlockSpec((1, _H1, _H2), lambda i, ts, ps, cn, b3: (ts[i], 0, 0)),
            pl.BlockSpec((1, 1, _H2), lambda i, ts, ps, cn, b3: (ts[i], 0, 0)),
            pl.BlockSpec((1, 1, _H2), lambda i, ts, ps, cn, b3: (ts[i], 0, 0)),
        ],
        out_specs=pl.BlockSpec(memory_space=pltpu.SMEM),
    )
    return pl.pallas_call(
        _mlp_body,
        grid_spec=grid_spec,
        out_shape=jax.ShapeDtypeStruct((1, 1), jnp.float32),
    )(ts, pstart, counts, b3.reshape(_S),
      xg, W1, b1.reshape(_S, 1, _H1), W2, b2.reshape(_S, 1, _H2),
      W3.reshape(_S, 1, _H2))


def kernel(aev, species, W1, b1, W2, b2, W3, b3):
    ts, pstart, counts, dest = _routing(species)
    xg = _sc_scatter(aev, dest)
    total = _mlp_call(xg, ts, pstart, counts, W1, b1, W2, b2, W3, b3)
    return total.reshape(1)


# speedup vs baseline: 2.7608x; 1.0086x over previous
import functools

import jax
import jax.numpy as jnp
from jax import lax
from jax.experimental import pallas as pl
from jax.experimental.pallas import tpu as pltpu
from jax.experimental.pallas import tpu_sc as plsc

_S = 8
_D = 1024
_H1 = 512
_H2 = 256
_N = 8192

_TILE = 256
_NT = _N // _TILE + _S
_NPAD = _NT * _TILE

_NC = 2
_NS = 16
_NW = _NC * _NS
_BPW = _N // _NW
_CHUNK = 32
_NCH = _BPW // _CHUNK

_BLK = 128
_NB = _N // _BLK


def _routing(species):
    species = species.astype(jnp.int32)
    oh = (species[:, None] == jnp.arange(_S, dtype=jnp.int32)[None, :])
    ohf = oh.astype(jnp.float32)
    ohb = ohf.reshape(_NB, _BLK, _S)
    ltri = jnp.tril(jnp.ones((_BLK, _BLK), jnp.float32))
    within = jnp.einsum('ij,bjs->bis', ltri, ohb,
                        preferred_element_type=jnp.float32)
    bsum = jnp.sum(ohb, axis=1)
    bpref = jnp.cumsum(bsum, axis=0) - bsum
    ranks = (within + bpref[:, None, :]).reshape(_N, _S)
    counts = (bpref[-1] + bsum[-1]).astype(jnp.int32)
    myrank = jnp.sum(ranks * ohf, axis=1) - 1.0
    ptiles = (counts + _TILE - 1) // _TILE
    pstart = (jnp.cumsum(ptiles) - ptiles).astype(jnp.int32)
    t = jnp.arange(_NT, dtype=jnp.int32)
    ts = jnp.sum((t[:, None] >= pstart[None, :]).astype(jnp.int32), axis=1) - 1
    ts = jnp.clip(ts, 0, _S - 1).astype(jnp.int32)
    seg0 = jnp.sum(ohf * pstart.astype(jnp.float32)[None, :], axis=1)
    dest = (seg0 * _TILE + myrank).astype(jnp.int32)
    return ts, pstart, counts, dest


def _sc_scatter(aev, dest):
    mesh = plsc.VectorSubcoreMesh(core_axis_name="c", subcore_axis_name="s")

    @functools.partial(
        pl.kernel,
        out_type=jax.ShapeDtypeStruct((_NPAD, _D), jnp.float32),
        mesh=mesh,
        scratch_types=[
            pltpu.VMEM((_NCH, _CHUNK), jnp.int32),
            pltpu.VMEM((3, _CHUNK, _D), jnp.float32),
            pltpu.SemaphoreType.DMA,
            pltpu.SemaphoreType.DMA,
            pltpu.SemaphoreType.DMA,
            pltpu.SemaphoreType.DMA,
            pltpu.SemaphoreType.DMA,
            pltpu.SemaphoreType.DMA,
        ],
    )
    def scatter_kernel(aev_hbm, dest_hbm, out_hbm, idx_v, rows_v,
                       g0, g1, g2, s0, s1, s2):
        gsems = (g0, g1, g2)
        ssems = (s0, s1, s2)
        wid = lax.axis_index("s") * _NC + lax.axis_index("c")
        pltpu.sync_copy(dest_hbm.at[wid], idx_v)
        base = wid * _BPW
        rd = [None] * _NCH
        wr = [None] * _NCH
        for c in range(min(2, _NCH)):
            rd[c] = pltpu.async_copy(
                aev_hbm.at[pl.ds(base + c * _CHUNK, _CHUNK)],
                rows_v.at[c % 3], gsems[c % 3])
        for c in range(_NCH):
            rd[c].wait()
            wr[c] = pltpu.async_copy(
                rows_v.at[c % 3], out_hbm.at[idx_v.at[c]], ssems[c % 3])
            if c + 2 < _NCH:
                if c >= 1:
                    wr[c - 1].wait()
                rd[c + 2] = pltpu.async_copy(
                    aev_hbm.at[pl.ds(base + (c + 2) * _CHUNK, _CHUNK)],
                    rows_v.at[(c + 2) % 3], gsems[(c + 2) % 3])
        for c in range(max(_NCH - 3, 0), _NCH):
            wr[c].wait()

    return scatter_kernel(aev, dest.reshape(_NW, _NCH, _CHUNK))


def _mlp_body(ts_ref, pstart_ref, counts_ref, b3_ref,
              x_ref, W1_ref, b1_ref, W2_ref, b2_ref, w3_ref, out_ref):
    i = pl.program_id(0)
    s = ts_ref[i]
    row = lax.broadcasted_iota(jnp.int32, (_TILE, 1), 0)
    off0 = (i - pstart_ref[s]) * _TILE
    valid_b = (row + off0) < counts_ref[s]
    x = jnp.where(valid_b, x_ref[...], 0.0)
    h = jnp.dot(x, W1_ref[s], preferred_element_type=jnp.float32)
    h = h + b1_ref[s]
    h = jnp.where(h > 0.0, h, jnp.exp(h) - 1.0)
    h = jnp.dot(h, W2_ref[s], preferred_element_type=jnp.float32)
    h = h + b2_ref[s]
    h = jnp.where(h > 0.0, h, jnp.exp(h) - 1.0)
    valid = valid_b.astype(jnp.float32)
    hsum = jnp.sum(h * valid, axis=0, keepdims=True)
    tile_e = jnp.sum(hsum * w3_ref[s]) + b3_ref[s] * jnp.sum(valid)

    @pl.when(i == 0)
    def _init():
        out_ref[0, 0] = 0.0

    out_ref[0, 0] += tile_e


def _mlp_call(xg, ts, pstart, counts, W1, b1, W2, b2, W3, b3):
    grid_spec = pltpu.PrefetchScalarGridSpec(
        num_scalar_prefetch=4,
        grid=(_NT,),
        in_specs=[
            pl.BlockSpec((_TILE, _D), lambda i, ts, ps, cn, b3: (i, 0)),
            pl.BlockSpec((_S, _D, _H1), lambda i, ts, ps, cn, b3: (0, 0, 0)),
            pl.BlockSpec((_S, 1, _H1), lambda i, ts, ps, cn, b3: (0, 0, 0)),
            pl.BlockSpec((_S, _H1, _H2), lambda i, ts, ps, cn, b3: (0, 0, 0)),
            pl.BlockSpec((_S, 1, _H2), lambda i, ts, ps, cn, b3: (0, 0, 0)),
            pl.BlockSpec((_S, 1, _H2), lambda i, ts, ps, cn, b3: (0, 0, 0)),
        ],
        out_specs=pl.BlockSpec(memory_space=pltpu.SMEM),
    )
    return pl.pallas_call(
        _mlp_body,
        grid_spec=grid_spec,
        out_shape=jax.ShapeDtypeStruct((1, 1), jnp.float32),
    )(ts, pstart, counts, b3.reshape(_S),
      xg, W1, b1.reshape(_S, 1, _H1), W2, b2.reshape(_S, 1, _H2),
      W3.reshape(_S, 1, _H2))


def kernel(aev, species, W1, b1, W2, b2, W3, b3):
    ts, pstart, counts, dest = _routing(species)
    xg = _sc_scatter(aev, dest)
    total = _mlp_call(xg, ts, pstart, counts, W1, b1, W2, b2, W3, b3)
    return total.reshape(1)


# speedup vs baseline: 2.7611x; 1.0001x over previous
import functools

import jax
import jax.numpy as jnp
from jax import lax
from jax.experimental import pallas as pl
from jax.experimental.pallas import tpu as pltpu
from jax.experimental.pallas import tpu_sc as plsc

_S = 8
_D = 1024
_H1 = 512
_H2 = 256
_N = 8192

_TILE = 256
_NT = _N // _TILE + _S
_NPAD = _NT * _TILE

_NC = 2
_NS = 16
_NW = _NC * _NS
_BPW = _N // _NW
_CHUNK = 32
_NCH = _BPW // _CHUNK

_BLK = 128
_NB = _N // _BLK


def _routing(species):
    species = species.astype(jnp.int32)
    oh = (species[:, None] == jnp.arange(_S, dtype=jnp.int32)[None, :])
    ohf = oh.astype(jnp.float32)
    ohb = ohf.reshape(_NB, _BLK, _S)
    ltri = jnp.tril(jnp.ones((_BLK, _BLK), jnp.float32))
    within = jnp.einsum('ij,bjs->bis', ltri, ohb,
                        preferred_element_type=jnp.float32)
    bsum = jnp.sum(ohb, axis=1)
    bpref = jnp.cumsum(bsum, axis=0) - bsum
    ranks = (within + bpref[:, None, :]).reshape(_N, _S)
    counts = (bpref[-1] + bsum[-1]).astype(jnp.int32)
    myrank = jnp.sum(ranks * ohf, axis=1) - 1.0
    ptiles = (counts + _TILE - 1) // _TILE
    pstart = (jnp.cumsum(ptiles) - ptiles).astype(jnp.int32)
    t = jnp.arange(_NT, dtype=jnp.int32)
    ts = jnp.sum((t[:, None] >= pstart[None, :]).astype(jnp.int32), axis=1) - 1
    ts = jnp.clip(ts, 0, _S - 1).astype(jnp.int32)
    seg0 = jnp.sum(ohf * pstart.astype(jnp.float32)[None, :], axis=1)
    dest = (seg0 * _TILE + myrank).astype(jnp.int32)
    return ts, pstart, counts, dest


def _sc_scatter(aev, dest):
    mesh = plsc.VectorSubcoreMesh(core_axis_name="c", subcore_axis_name="s")

    @functools.partial(
        pl.kernel,
        out_type=jax.ShapeDtypeStruct((_NPAD, _D), jnp.float32),
        mesh=mesh,
        scratch_types=[
            pltpu.VMEM((_NCH, _CHUNK), jnp.int32),
            pltpu.VMEM((3, _CHUNK, _D), jnp.float32),
            pltpu.SemaphoreType.DMA,
            pltpu.SemaphoreType.DMA,
            pltpu.SemaphoreType.DMA,
            pltpu.SemaphoreType.DMA,
            pltpu.SemaphoreType.DMA,
            pltpu.SemaphoreType.DMA,
        ],
    )
    def scatter_kernel(aev_hbm, dest_hbm, out_hbm, idx_v, rows_v,
                       g0, g1, g2, s0, s1, s2):
        gsems = (g0, g1, g2)
        ssems = (s0, s1, s2)
        wid = lax.axis_index("s") * _NC + lax.axis_index("c")
        pltpu.sync_copy(dest_hbm.at[wid], idx_v)
        base = wid * _BPW
        rd = [None] * _NCH
        wr = [None] * _NCH
        for c in range(min(2, _NCH)):
            rd[c] = pltpu.async_copy(
                aev_hbm.at[pl.ds(base + c * _CHUNK, _CHUNK)],
                rows_v.at[c % 3], gsems[c % 3])
        for c in range(_NCH):
            rd[c].wait()
            wr[c] = pltpu.async_copy(
                rows_v.at[c % 3], out_hbm.at[idx_v.at[c]], ssems[c % 3])
            if c + 2 < _NCH:
                if c >= 1:
                    wr[c - 1].wait()
                rd[c + 2] = pltpu.async_copy(
                    aev_hbm.at[pl.ds(base + (c + 2) * _CHUNK, _CHUNK)],
                    rows_v.at[(c + 2) % 3], gsems[(c + 2) % 3])
        for c in range(max(_NCH - 3, 0), _NCH):
            wr[c].wait()

    return scatter_kernel(aev, dest.reshape(_NW, _NCH, _CHUNK))


def _mlp_body(ts_ref, pstart_ref, counts_ref, b3_ref,
              xl_ref, xr_ref, W1_ref, b1_ref, W2_ref, b2_ref, w3_ref,
              out_ref):
    i = pl.program_id(0)
    s = ts_ref[i]
    row = lax.broadcasted_iota(jnp.int32, (_TILE, 1), 0)
    off0 = (i - pstart_ref[s]) * _TILE
    valid_b = (row + off0) < counts_ref[s]
    xl = jnp.where(valid_b, xl_ref[...], 0.0)
    xr = jnp.where(valid_b, xr_ref[...], 0.0)
    h = (jnp.dot(xl, W1_ref[s, :_D // 2], preferred_element_type=jnp.float32)
         + jnp.dot(xr, W1_ref[s, _D // 2:], preferred_element_type=jnp.float32))
    h = h + b1_ref[s]
    h = jnp.where(h > 0.0, h, jnp.exp(h) - 1.0)
    h = jnp.dot(h, W2_ref[s], preferred_element_type=jnp.float32)
    h = h + b2_ref[s]
    h = jnp.where(h > 0.0, h, jnp.exp(h) - 1.0)
    valid = valid_b.astype(jnp.float32)
    hsum = jnp.sum(h * valid, axis=0, keepdims=True)
    tile_e = jnp.sum(hsum * w3_ref[s]) + b3_ref[s] * jnp.sum(valid)

    @pl.when(i == 0)
    def _init():
        out_ref[0, 0] = 0.0

    out_ref[0, 0] += tile_e


def _mlp_call(xg, ts, pstart, counts, W1, b1, W2, b2, W3, b3):
    grid_spec = pltpu.PrefetchScalarGridSpec(
        num_scalar_prefetch=4,
        grid=(_NT,),
        in_specs=[
            pl.BlockSpec((_TILE, _D // 2), lambda i, ts, ps, cn, b3: (i, 0)),
            pl.BlockSpec((_TILE, _D // 2), lambda i, ts, ps, cn, b3: (i, 1)),
            pl.BlockSpec((_S, _D, _H1), lambda i, ts, ps, cn, b3: (0, 0, 0)),
            pl.BlockSpec((_S, 1, _H1), lambda i, ts, ps, cn, b3: (0, 0, 0)),
            pl.BlockSpec((_S, _H1, _H2), lambda i, ts, ps, cn, b3: (0, 0, 0)),
            pl.BlockSpec((_S, 1, _H2), lambda i, ts, ps, cn, b3: (0, 0, 0)),
            pl.BlockSpec((_S, 1, _H2), lambda i, ts, ps, cn, b3: (0, 0, 0)),
        ],
        out_specs=pl.BlockSpec(memory_space=pltpu.SMEM),
    )
    return pl.pallas_call(
        _mlp_body,
        grid_spec=grid_spec,
        out_shape=jax.ShapeDtypeStruct((1, 1), jnp.float32),
    )(ts, pstart, counts, b3.reshape(_S),
      xg, xg, W1, b1.reshape(_S, 1, _H1), W2, b2.reshape(_S, 1, _H2),
      W3.reshape(_S, 1, _H2))


def kernel(aev, species, W1, b1, W2, b2, W3, b3):
    ts, pstart, counts, dest = _routing(species)
    xg = _sc_scatter(aev, dest)
    total = _mlp_call(xg, ts, pstart, counts, W1, b1, W2, b2, W3, b3)
    return total.reshape(1)
